# baseline (speedup 1.0000x reference)
import jax
import jax.numpy as jnp
from jax import lax
from jax.experimental import pallas as pl
from jax.experimental.pallas import tpu as pltpu
from jax.experimental.pallas import tpu_sc as plsc

BATCH = 16384
DIM = 64
NC = 2
NS = 16
NW = NC * NS
BPW = BATCH // NW


def _gather_body(users_hbm, neg_hbm, user_table_hbm, entity_table_hbm,
                 u_out, n_out, uidx_v, nidx_v, urows_v, nrows_v, sem1, sem2):
    wid = lax.axis_index("s") * NC + lax.axis_index("c")
    base = wid * BPW
    pltpu.sync_copy(users_hbm.at[pl.ds(base, BPW)], uidx_v)
    pltpu.sync_copy(neg_hbm.at[pl.ds(base, BPW)], nidx_v)
    c1 = pltpu.async_copy(user_table_hbm.at[uidx_v], urows_v, sem1)
    c2 = pltpu.async_copy(entity_table_hbm.at[nidx_v], nrows_v, sem2)
    c1.wait()
    c2.wait()
    pltpu.sync_copy(urows_v, u_out.at[pl.ds(base, BPW)])
    pltpu.sync_copy(nrows_v, n_out.at[pl.ds(base, BPW)])


_gather = pl.kernel(
    _gather_body,
    mesh=plsc.VectorSubcoreMesh(core_axis_name="c", subcore_axis_name="s"),
    out_type=(
        jax.ShapeDtypeStruct((BATCH, DIM), jnp.float32),
        jax.ShapeDtypeStruct((BATCH, DIM), jnp.float32),
    ),
    scratch_types=[
        pltpu.VMEM((BPW,), jnp.int32),
        pltpu.VMEM((BPW,), jnp.int32),
        pltpu.VMEM((BPW, DIM), jnp.float32),
        pltpu.VMEM((BPW, DIM), jnp.float32),
        pltpu.SemaphoreType.DMA,
        pltpu.SemaphoreType.DMA,
    ],
    compiler_params=pltpu.CompilerParams(use_tc_tiling_on_sc=False),
)


def _loss_body(u_ref, n_ref, out_ref):
    z = u_ref[...] * n_ref[...]
    sp = jnp.maximum(z, 0.0) + jnp.log1p(jnp.exp(-jnp.abs(z)))
    out_ref[0, 0] = jnp.mean(sp)


def kernel(users, pos, neg, user_table, entity_table):
    del pos
    u_g, n_g = _gather(users.astype(jnp.int32), neg.astype(jnp.int32),
                       user_table, entity_table)
    loss = pl.pallas_call(
        _loss_body,
        out_shape=jax.ShapeDtypeStruct((1, 1), jnp.float32),
        out_specs=pl.BlockSpec(memory_space=pltpu.SMEM),
    )(u_g, n_g)
    return loss[0, 0]


# speedup vs baseline: 1.6457x; 1.6457x over previous
import jax
import jax.numpy as jnp
from jax import lax
from jax.experimental import pallas as pl
from jax.experimental.pallas import tpu as pltpu
from jax.experimental.pallas import tpu_sc as plsc

BATCH = 16384
DIM = 64
NC = 2
NS = 16
NW = NC * NS
BPW = BATCH // NW
R = 64
CH = BPW // R


def _gather_body(users_hbm, neg_hbm, user_table_hbm, entity_table_hbm,
                 u_out, n_out, uidx_v, nidx_v,
                 ubuf_a, ubuf_b, nbuf_a, nbuf_b,
                 sem_dma_a, sem_dma_b, sem_out_a, sem_out_b):
    wid = lax.axis_index("s") * NC + lax.axis_index("c")
    base = wid * BPW
    pltpu.sync_copy(users_hbm.at[pl.ds(base, BPW)], uidx_v)
    pltpu.sync_copy(neg_hbm.at[pl.ds(base, BPW)], nidx_v)

    ubufs, nbufs = (ubuf_a, ubuf_b), (nbuf_a, nbuf_b)
    sem_dma, sem_out = (sem_dma_a, sem_dma_b), (sem_out_a, sem_out_b)

    def issue(c, p):
        ub, nb, sem = ubufs[p], nbufs[p], sem_dma[p]

        def b(g, carry):
            uvec = uidx_v[pl.ds(g * 16, 16)]
            nvec = nidx_v[pl.ds(g * 16, 16)]
            slot0 = (g - c * (R // 16)) * 16
            for k in range(16):
                pltpu.async_copy(user_table_hbm.at[pl.ds(uvec[k], 1)],
                                 ub.at[pl.ds(slot0 + k, 1)], sem)
                pltpu.async_copy(entity_table_hbm.at[pl.ds(nvec[k], 1)],
                                 nb.at[pl.ds(slot0 + k, 1)], sem)
            return carry

        lax.fori_loop(c * (R // 16), (c + 1) * (R // 16), b, 0)

    def drain_and_flush(c, p):
        ub, nb = ubufs[p], nbufs[p]
        pltpu.make_async_copy(user_table_hbm.at[pl.ds(0, R)], ub,
                              sem_dma[p]).wait()
        pltpu.make_async_copy(entity_table_hbm.at[pl.ds(0, R)], nb,
                              sem_dma[p]).wait()
        pltpu.async_copy(ub, u_out.at[pl.ds(base + c * R, R)], sem_out[p])
        pltpu.async_copy(nb, n_out.at[pl.ds(base + c * R, R)], sem_out[p])

    def wait_flush(p):
        pltpu.make_async_copy(user_table_hbm.at[pl.ds(0, R)], ubufs[p],
                              sem_out[p]).wait()
        pltpu.make_async_copy(entity_table_hbm.at[pl.ds(0, R)], nbufs[p],
                              sem_out[p]).wait()

    for c in range(CH):
        p = c % 2
        if c >= 2:
            wait_flush(p)
        issue(c, p)
        if c >= 1:
            drain_and_flush(c - 1, 1 - p)
    drain_and_flush(CH - 1, (CH - 1) % 2)
    wait_flush(0)
    wait_flush(1)


_gather = pl.kernel(
    _gather_body,
    mesh=plsc.VectorSubcoreMesh(core_axis_name="c", subcore_axis_name="s"),
    out_type=(
        jax.ShapeDtypeStruct((BATCH, DIM), jnp.float32),
        jax.ShapeDtypeStruct((BATCH, DIM), jnp.float32),
    ),
    scratch_types=[
        pltpu.VMEM((BPW,), jnp.int32),
        pltpu.VMEM((BPW,), jnp.int32),
        pltpu.VMEM((R, DIM), jnp.float32),
        pltpu.VMEM((R, DIM), jnp.float32),
        pltpu.VMEM((R, DIM), jnp.float32),
        pltpu.VMEM((R, DIM), jnp.float32),
        pltpu.SemaphoreType.DMA,
        pltpu.SemaphoreType.DMA,
        pltpu.SemaphoreType.DMA,
        pltpu.SemaphoreType.DMA,
    ],
)


def _loss_body(u_ref, n_ref, out_ref):
    z = u_ref[...] * n_ref[...]
    sp = jnp.maximum(z, 0.0) + jnp.log1p(jnp.exp(-jnp.abs(z)))
    out_ref[0, 0] = jnp.mean(sp)


def kernel(users, pos, neg, user_table, entity_table):
    del pos
    u_g, n_g = _gather(users.astype(jnp.int32), neg.astype(jnp.int32),
                       user_table, entity_table)
    loss = pl.pallas_call(
        _loss_body,
        out_shape=jax.ShapeDtypeStruct((1, 1), jnp.float32),
        out_specs=pl.BlockSpec(memory_space=pltpu.SMEM),
    )(u_g, n_g)
    return loss[0, 0]
